# baseline (speedup 1.0000x reference)
import jax
import jax.numpy as jnp
from jax.experimental import pallas as pl

B, C = 1024, 7
CC = C * C
BI = 16


def _body(x49_ref, xrep_ref, w49_ref, out_ref):
    x = x49_ref[...]
    w = w49_ref[...]
    p0 = jnp.abs(x) * w
    d = (jnp.abs(x - 1.0) - jnp.abs(x)) * w
    xr = xrep_ref[...]
    idx = (xr * (xr >= 0).astype(xr.dtype)).astype(jnp.int32)
    kio = jax.lax.broadcasted_iota(jnp.int32, (BI, CC), 1) % C
    m = (idx == kio).astype(jnp.float32)
    out_ref[...] = p0[None, :, :] + m[:, None, :] * d[None, :, :]


def kernel(input, target, code_weights):
    x = input.reshape(B, C)
    x49 = jnp.tile(x, (1, C))
    xrep = jnp.repeat(x, C, axis=1)
    w49 = jnp.tile(code_weights, C).reshape(1, CC)

    out = pl.pallas_call(
        _body,
        grid=(B // BI,),
        in_specs=[
            pl.BlockSpec((B, CC), lambda i: (0, 0)),
            pl.BlockSpec((BI, CC), lambda i: (i, 0)),
            pl.BlockSpec((1, CC), lambda i: (0, 0)),
        ],
        out_specs=pl.BlockSpec((BI, B, CC), lambda i: (i, 0, 0)),
        out_shape=jax.ShapeDtypeStruct((B, B, CC), jnp.float32),
    )(x49, xrep, w49)
    return out.reshape(B, B, C, C)
